# baseline (speedup 1.0000x reference)
import functools

import jax
import jax.numpy as jnp
from jax.experimental import pallas as pl

_BN_EPS = 1e-5


def _fps_kernel(xT_ref, oi_ref, oc_ref, *, npoint, N, R, C):
    xT = xT_ref[...]
    lane3 = jax.lax.broadcasted_iota(jnp.int32, (1, 1, N), 2)
    lane2 = jax.lax.broadcasted_iota(jnp.int32, (1, N), 1)
    slot = (jax.lax.broadcasted_iota(jnp.int32, (1, R, C), 1) * C
            + jax.lax.broadcasted_iota(jnp.int32, (1, R, C), 2))
    lane_np = jax.lax.broadcasted_iota(jnp.int32, (1, 1, npoint), 2)

    def body(i, carry):
        distance, f, acc, accc = carry
        acc = jnp.where(slot == i, f[:, :, None], acc)
        c = jnp.sum(jnp.where(lane3 == f[:, :, None], xT, 0.0),
                    axis=2, keepdims=True)
        accc = jnp.where(lane_np == i, c, accc)
        diff = xT - c
        dist = jnp.sum(diff * diff, axis=1)
        distance = jnp.minimum(distance, dist)
        m = jnp.max(distance, axis=1, keepdims=True)
        f2 = jnp.min(jnp.where(distance == m, lane2, N),
                     axis=1, keepdims=True).astype(jnp.int32)
        return distance, f2, acc, accc

    B = xT.shape[0]
    init = (jnp.full((B, N), 1e10, jnp.float32),
            jnp.zeros((B, 1), jnp.int32),
            jnp.zeros((B, R, C), jnp.int32),
            jnp.zeros((B, 3, npoint), jnp.float32))
    _, _, acc, accc = jax.lax.fori_loop(0, npoint, body, init)
    oi_ref[...] = acc
    oc_ref[...] = accc


def _fps_pallas(xyz, npoint):
    B, N, _ = xyz.shape
    xT = jnp.transpose(xyz, (0, 2, 1))
    if npoint >= 128:
        R, C = npoint // 128, 128
    else:
        R, C = 1, npoint
    idx, cT = pl.pallas_call(
        functools.partial(_fps_kernel, npoint=npoint, N=N, R=R, C=C),
        out_shape=[
            jax.ShapeDtypeStruct((B, R, C), jnp.int32),
            jax.ShapeDtypeStruct((B, 3, npoint), jnp.float32),
        ],
    )(xT)
    return idx.reshape(B, npoint), jnp.transpose(cT, (0, 2, 1))


def _ballq_kernel(xT_ref, c_ref, o_ref, *, r2, nsample, N):
    xT = xT_ref[0]
    c = c_ref[0]
    d = -2.0 * jnp.dot(c, xT, preferred_element_type=jnp.float32)
    d = d + jnp.sum(c * c, axis=1, keepdims=True)
    d = d + jnp.sum(xT * xT, axis=0, keepdims=True)
    lane = jax.lax.broadcasted_iota(jnp.int32, d.shape, 1)
    cand = jnp.where(d > r2, N, lane)
    first = jnp.min(cand, axis=1, keepdims=True)
    cur = cand
    cols = []
    for _ in range(nsample):
        jk = jnp.min(cur, axis=1, keepdims=True)
        cols.append(jnp.where(jk == N, first, jk))
        cur = jnp.where(cur == jk, N, cur)
    o_ref[0] = jnp.concatenate(cols, axis=1)


def _ball_query_pallas(radius, nsample, xyz, new_xyz):
    B, N, _ = xyz.shape
    S = new_xyz.shape[1]
    TS = min(S, 256)
    xT = jnp.transpose(xyz, (0, 2, 1))
    return pl.pallas_call(
        functools.partial(_ballq_kernel, r2=radius ** 2, nsample=nsample, N=N),
        grid=(B, S // TS),
        in_specs=[
            pl.BlockSpec((1, 3, N), lambda b, s: (b, 0, 0)),
            pl.BlockSpec((1, TS, 3), lambda b, s: (b, s, 0)),
        ],
        out_specs=pl.BlockSpec((1, TS, nsample), lambda b, s: (b, s, 0)),
        out_shape=jax.ShapeDtypeStruct((B, S, nsample), jnp.int32),
    )(xT, new_xyz)


def _knn_kernel(xT_ref, c_ref, od_ref, oi_ref, *, k, N):
    xT = xT_ref[0]
    c = c_ref[0]
    d = -2.0 * jnp.dot(c, xT, preferred_element_type=jnp.float32)
    d = d + jnp.sum(c * c, axis=1, keepdims=True)
    d = d + jnp.sum(xT * xT, axis=0, keepdims=True)
    lane = jax.lax.broadcasted_iota(jnp.int32, d.shape, 1)
    cur = d
    dcols, icols = [], []
    for _ in range(k):
        m = jnp.min(cur, axis=1, keepdims=True)
        a = jnp.min(jnp.where(cur == m, lane, N), axis=1, keepdims=True)
        dcols.append(m)
        icols.append(a)
        cur = jnp.where(lane == a, jnp.float32(jnp.inf), cur)
    od_ref[0] = jnp.concatenate(dcols, axis=1)
    oi_ref[0] = jnp.concatenate(icols, axis=1)


def _knn_pallas(k, xyz, new_xyz):
    B, N, _ = xyz.shape
    S = new_xyz.shape[1]
    TS = min(S, 256)
    xT = jnp.transpose(xyz, (0, 2, 1))
    return pl.pallas_call(
        functools.partial(_knn_kernel, k=k, N=N),
        grid=(B, S // TS),
        in_specs=[
            pl.BlockSpec((1, 3, N), lambda b, s: (b, 0, 0)),
            pl.BlockSpec((1, TS, 3), lambda b, s: (b, s, 0)),
        ],
        out_specs=[
            pl.BlockSpec((1, TS, k), lambda b, s: (b, s, 0)),
            pl.BlockSpec((1, TS, k), lambda b, s: (b, s, 0)),
        ],
        out_shape=[
            jax.ShapeDtypeStruct((B, S, k), jnp.float32),
            jax.ShapeDtypeStruct((B, S, k), jnp.int32),
        ],
    )(xT, new_xyz)


def _gather_kernel(t_ref, i_ref, o_ref, *, k, N):
    T = t_ref[0]
    idxt = i_ref[0]
    lane = jax.lax.broadcasted_iota(jnp.int32, (idxt.shape[0], N), 1)
    rows = []
    for j in range(k):
        col = jax.lax.slice_in_dim(idxt, j, j + 1, axis=1)
        oh = (lane == col).astype(jnp.float32)
        rows.append(jnp.dot(oh, T, preferred_element_type=jnp.float32))
    o_ref[0] = jnp.stack(rows, axis=1)


def _gather_pallas(table, idx):
    B, N, C = table.shape
    S, k = idx.shape[1], idx.shape[2]
    budget = max(1, (2 * 1024 * 1024) // (k * max(C, 128) * 4))
    TS = min(S, 256)
    while TS > budget and TS > 8:
        TS //= 2
    return pl.pallas_call(
        functools.partial(_gather_kernel, k=k, N=N),
        grid=(B, S // TS),
        in_specs=[
            pl.BlockSpec((1, N, C), lambda b, s: (b, 0, 0)),
            pl.BlockSpec((1, TS, k), lambda b, s: (b, s, 0)),
        ],
        out_specs=pl.BlockSpec((1, TS, k, C), lambda b, s: (b, s, 0, 0)),
        out_shape=jax.ShapeDtypeStruct((B, S, k, C), jnp.float32),
    )(table, idx)


def _fp_interp_kernel(xT_ref, c_ref, f2_ref, o_ref, *, N):
    xT = xT_ref[0]
    c = c_ref[0]
    d = -2.0 * jnp.dot(c, xT, preferred_element_type=jnp.float32)
    d = d + jnp.sum(c * c, axis=1, keepdims=True)
    d = d + jnp.sum(xT * xT, axis=0, keepdims=True)
    lane = jax.lax.broadcasted_iota(jnp.int32, d.shape, 1)
    cur = d
    ms, as_ = [], []
    for _ in range(3):
        m = jnp.min(cur, axis=1, keepdims=True)
        a = jnp.min(jnp.where(cur == m, lane, N), axis=1, keepdims=True)
        ms.append(m)
        as_.append(a)
        cur = jnp.where(lane == a, jnp.float32(jnp.inf), cur)
    ws = [1.0 / jnp.maximum(m, 1e-10) for m in ms]
    wsum = ws[0] + ws[1] + ws[2]
    oh = jnp.zeros_like(d)
    for w, a in zip(ws, as_):
        oh = oh + jnp.where(lane == a, w / wsum, 0.0)
    o_ref[0] = jnp.dot(oh, f2_ref[0], preferred_element_type=jnp.float32)


def _fp_interp_pallas(pos1, pos2, feat2):
    B, N, _ = pos2.shape
    S = pos1.shape[1]
    C = feat2.shape[-1]
    TS = min(S, 512)
    xT = jnp.transpose(pos2, (0, 2, 1))
    return pl.pallas_call(
        functools.partial(_fp_interp_kernel, N=N),
        grid=(B, S // TS),
        in_specs=[
            pl.BlockSpec((1, 3, N), lambda b, s: (b, 0, 0)),
            pl.BlockSpec((1, TS, 3), lambda b, s: (b, s, 0)),
            pl.BlockSpec((1, N, C), lambda b, s: (b, 0, 0)),
        ],
        out_specs=pl.BlockSpec((1, TS, C), lambda b, s: (b, s, 0)),
        out_shape=jax.ShapeDtypeStruct((B, S, C), jnp.float32),
    )(xT, pos1, feat2)


def _mm_stats_kernel(x_ref, w_ref, y_ref, s_ref, ss_ref):
    i = pl.program_id(0)
    y = jnp.dot(x_ref[...], w_ref[...], preferred_element_type=jnp.float32)
    y_ref[...] = y

    @pl.when(i == 0)
    def _():
        s_ref[...] = jnp.zeros_like(s_ref)
        ss_ref[...] = jnp.zeros_like(ss_ref)

    s_ref[...] += jnp.sum(y, axis=0, keepdims=True)
    ss_ref[...] += jnp.sum(y * y, axis=0, keepdims=True)


def _matmul_stats(xf, W, tile_m):
    M, C = xf.shape
    Co = W.shape[1]
    grid = (M // tile_m,)
    return pl.pallas_call(
        _mm_stats_kernel,
        grid=grid,
        in_specs=[
            pl.BlockSpec((tile_m, C), lambda i: (i, 0)),
            pl.BlockSpec((C, Co), lambda i: (0, 0)),
        ],
        out_specs=[
            pl.BlockSpec((tile_m, Co), lambda i: (i, 0)),
            pl.BlockSpec((1, Co), lambda i: (0, 0)),
            pl.BlockSpec((1, Co), lambda i: (0, 0)),
        ],
        out_shape=[
            jax.ShapeDtypeStruct((M, Co), jnp.float32),
            jax.ShapeDtypeStruct((1, Co), jnp.float32),
            jax.ShapeDtypeStruct((1, Co), jnp.float32),
        ],
    )(xf, W)


def _bn_mm_kernel(y_ref, s_ref, ss_ref, g_ref, b_ref, w2_ref, b2_ref, o_ref, *, count):
    mean = s_ref[...] / count
    var = ss_ref[...] / count - mean * mean
    xn = jax.nn.relu(
        g_ref[...] * (y_ref[...] - mean) / jnp.sqrt(var + _BN_EPS) + b_ref[...]
    )
    o_ref[...] = (
        jnp.dot(xn, w2_ref[...], preferred_element_type=jnp.float32) + b2_ref[...]
    )


def _bn_relu_matmul(y, s, ss, g, b, W2, b2, tile_m):
    M, C = y.shape
    Co = W2.shape[1]
    grid = (M // tile_m,)
    return pl.pallas_call(
        functools.partial(_bn_mm_kernel, count=float(M)),
        grid=grid,
        in_specs=[
            pl.BlockSpec((tile_m, C), lambda i: (i, 0)),
            pl.BlockSpec((1, C), lambda i: (0, 0)),
            pl.BlockSpec((1, C), lambda i: (0, 0)),
            pl.BlockSpec((1, C), lambda i: (0, 0)),
            pl.BlockSpec((1, C), lambda i: (0, 0)),
            pl.BlockSpec((C, Co), lambda i: (0, 0)),
            pl.BlockSpec((1, Co), lambda i: (0, 0)),
        ],
        out_specs=pl.BlockSpec((tile_m, Co), lambda i: (i, 0)),
        out_shape=jax.ShapeDtypeStruct((M, Co), jnp.float32),
    )(y, s, ss, g.reshape(1, C), b.reshape(1, C), W2, b2.reshape(1, Co))


def _bn_relu(y, g, b, axes):
    mean = jnp.mean(y, axis=axes, keepdims=True)
    var = jnp.mean((y - mean) ** 2, axis=axes, keepdims=True)
    return jax.nn.relu(g * (y - mean) / jnp.sqrt(var + _BN_EPS) + b)


def _run_mlp(x, layers, axes):
    for (W, g, b) in layers:
        x = _bn_relu(jnp.matmul(x, W), g, b, axes)
    return x


def _set_abstraction(xyz, points, npoint, radius, nsample, layers):
    _, new_xyz = _fps_pallas(xyz, npoint)
    idx = _ball_query_pallas(radius, nsample, xyz, new_xyz)
    grouped = _gather_pallas(jnp.concatenate([xyz, points], -1), idx)
    grouped_xyz = grouped[..., :3] - new_xyz[:, :, None, :]
    new_points = jnp.concatenate([grouped_xyz, grouped[..., 3:]], -1)
    new_points = _run_mlp(new_points, layers, (0, 1, 2))
    return new_xyz, jnp.max(new_points, axis=2)


def _flow_embedding(pos1, pos2, feat1, feat2, nsample, layers):
    _, idx = _knn_pallas(nsample, pos2, pos1)
    grouped = _gather_pallas(jnp.concatenate([pos2, feat2], -1), idx)
    pos_diff = grouped[..., :3] - pos1[:, :, None, :]
    feat2_g = grouped[..., 3:]
    feat1_e = jnp.broadcast_to(feat1[:, :, None, :], feat2_g.shape)
    x = jnp.concatenate([pos_diff, feat2_g, feat1_e], -1)
    x = _run_mlp(x, layers, (0, 1, 2))
    return pos1, jnp.max(x, axis=2)


def _set_upconv(pos1, pos2, feat1, feat2, nsample, layers1, layers2):
    _, idx = _knn_pallas(nsample, pos2, pos1)
    C2 = feat2.shape[-1]
    grouped = _gather_pallas(jnp.concatenate([feat2, pos2], -1), idx)
    pos_diff = grouped[..., C2:] - pos1[:, :, None, :]
    x = jnp.concatenate([grouped[..., :C2], pos_diff], -1)
    x = _run_mlp(x, layers1, (0, 1, 2))
    x = jnp.max(x, axis=2)
    if feat1 is not None:
        x = jnp.concatenate([x, feat1], -1)
    x = _run_mlp(x, layers2, (0, 1))
    return x


def _feature_propagation(pos1, pos2, feat1, feat2, layers):
    interp = _fp_interp_pallas(pos1, pos2, feat2)
    x = jnp.concatenate([interp, feat1], -1)
    return _run_mlp(x, layers, (0, 1))


def kernel(pc1, pc2, feature1, feature2, params):
    l1_pc1, l1_f1 = _set_abstraction(pc1, feature1, 1024, 0.5, 16, params['sa1'])
    l2_pc1, l2_f1 = _set_abstraction(l1_pc1, l1_f1, 256, 1.0, 16, params['sa2'])
    l1_pc2, l1_f2 = _set_abstraction(pc2, feature2, 1024, 0.5, 16, params['sa1'])
    l2_pc2, l2_f2 = _set_abstraction(l1_pc2, l1_f2, 256, 1.0, 16, params['sa2'])
    _, l2_f1_new = _flow_embedding(l2_pc1, l2_pc2, l2_f1, l2_f2, 64, params['fe'])
    l3_pc1, l3_f1 = _set_abstraction(l2_pc1, l2_f1_new, 64, 2.0, 8, params['sa3'])
    l4_pc1, l4_f1 = _set_abstraction(l3_pc1, l3_f1, 16, 4.0, 8, params['sa4'])
    l3_fnew = _set_upconv(l3_pc1, l4_pc1, l3_f1, l4_f1, 8,
                          params['su1_mlp'], params['su1_mlp2'])
    l2_fnew = _set_upconv(l2_pc1, l3_pc1,
                          jnp.concatenate([l2_f1, l2_f1_new], -1), l3_fnew, 8,
                          params['su2_mlp'], params['su2_mlp2'])
    l1_fnew = _set_upconv(l1_pc1, l2_pc1, l1_f1, l2_fnew, 8,
                          params['su3_mlp'], params['su3_mlp2'])
    l0_fnew = _feature_propagation(pc1, l1_pc1, feature1, l1_fnew, params['fp'])

    B, N, C = l0_fnew.shape
    xf = l0_fnew.reshape(B * N, C)
    W1, g1, b1 = params['head1']
    W2, b2 = params['head2']
    y, s, ss = _matmul_stats(xf, W1, tile_m=2048)
    sf = _bn_relu_matmul(y, s, ss, g1, b1, W2, b2, tile_m=2048)
    sf = sf.reshape(B, N, W2.shape[1])
    return jnp.transpose(sf, (0, 2, 1))
